# baseline (speedup 1.0000x reference)
import functools

import jax
import jax.numpy as jnp
from jax import lax
from jax.experimental import pallas as pl
from jax.experimental.pallas import tpu as pltpu
from jax.experimental.pallas import tpu_sc as plsc

N_NODES = 50000
N_EDGES = 800000

NC = 2
NS = 16
NW = NC * NS

CHUNK = 128
E_PER_TILE_CHUNKS = 200
E_PER_TILE = E_PER_TILE_CHUNKS * CHUNK
E_PAD = E_PER_TILE * NW
N_P = 51200
ROWS_PER_TILE = N_P // NS
F = 32

_mesh = plsc.VectorSubcoreMesh(core_axis_name="c", subcore_axis_name="s")


DEG_K = 8


def _deg_body(dst2d, ones_hbm, zeros_hbm, out, dst_v, ones_v, acc, sem):
    cid = lax.axis_index("c")
    sid = lax.axis_index("s")
    w = cid * NS + sid
    pltpu.sync_copy(dst2d.at[w], dst_v)
    pltpu.sync_copy(ones_hbm, ones_v)
    pltpu.sync_copy(zeros_hbm, acc.at[pl.ds(sid * ROWS_PER_TILE, ROWS_PER_TILE)])
    plsc.subcore_barrier()

    def body(g, carry):
        for i in range(DEG_K):
            pltpu.async_copy(ones_v, acc.at[dst_v.at[DEG_K * g + i]], sem,
                             add=True)
        for i in range(DEG_K):
            pltpu.make_async_copy(ones_v, acc.at[dst_v.at[0]], sem).wait()
        return carry

    lax.fori_loop(0, E_PER_TILE_CHUNKS // DEG_K, body, 0, unroll=False)
    plsc.subcore_barrier()
    pltpu.sync_copy(acc.at[pl.ds(sid * ROWS_PER_TILE, ROWS_PER_TILE)],
                    out.at[cid, pl.ds(sid * ROWS_PER_TILE, ROWS_PER_TILE)])


_deg_kernel = pl.kernel(
    _deg_body,
    out_type=jax.ShapeDtypeStruct((NC, N_P), jnp.float32),
    mesh=_mesh,
    scratch_types=[
        pltpu.VMEM((E_PER_TILE_CHUNKS, CHUNK), jnp.int32),
        pltpu.VMEM((CHUNK,), jnp.float32),
        pltpu.VMEM_SHARED((N_P,), jnp.float32),
        pltpu.SemaphoreType.DMA,
    ],
)


SUB = 20
SUP = E_PER_TILE_CHUNKS // SUB
NBUF = 4
NZC = ROWS_PER_TILE // CHUNK


def _agg_body(src2d, dst2d, hs, zeros_hbm, out,
              src_v, dst_v, bufs, acc, gsems, ssems, isems):
    cid = lax.axis_index("c")
    sid = lax.axis_index("s")
    w = cid * NS + sid

    def istart(s, slab):
        pltpu.async_copy(src2d.at[w, pl.ds(s * SUB, SUB)], src_v.at[slab],
                         isems.at[slab])
        pltpu.async_copy(dst2d.at[w, pl.ds(s * SUB, SUB)], dst_v.at[slab],
                         isems.at[slab])

    def iwait(slab):
        pltpu.make_async_copy(src2d.at[w, pl.ds(0, SUB)], src_v.at[slab],
                              isems.at[slab]).wait()
        pltpu.make_async_copy(dst2d.at[w, pl.ds(0, SUB)], dst_v.at[slab],
                              isems.at[slab]).wait()

    istart(0, 0)
    pltpu.sync_copy(zeros_hbm,
                    acc.at[pl.ds(sid * ROWS_PER_TILE, ROWS_PER_TILE)])
    plsc.subcore_barrier()

    def gstart(slab, j, i):
        pltpu.async_copy(hs.at[src_v.at[slab, j]], bufs.at[i], gsems.at[i])

    def gwait(i):
        pltpu.make_async_copy(hs.at[src_v.at[0, 0]], bufs.at[i],
                              gsems.at[i]).wait()

    def sstart(slab, j, i):
        pltpu.async_copy(bufs.at[i], acc.at[dst_v.at[slab, j]],
                         ssems.at[i], add=True)

    def swait(i):
        pltpu.make_async_copy(bufs.at[i], acc.at[dst_v.at[0, 0]],
                              ssems.at[i]).wait()

    def super_body(s, carry):
        slab = s % 2
        iwait(slab)

        @pl.when(s + 1 < SUP)
        def _():
            istart(s + 1, 1 - slab)

        for i in range(NBUF):
            gstart(slab, i, i)

        def body(g, carry2):
            j = NBUF * g
            for i in range(NBUF):
                gwait(i)
                sstart(slab, j + i, i)
            for i in range(NBUF):
                swait(i)
                gstart(slab, j + NBUF + i, i)
            return carry2

        lax.fori_loop(0, SUB // NBUF - 1, body, 0, unroll=False)
        j = SUB - NBUF
        for i in range(NBUF):
            gwait(i)
            sstart(slab, j + i, i)
        for i in range(NBUF):
            swait(i)
        return carry

    lax.fori_loop(0, SUP, super_body, 0, unroll=False)

    plsc.subcore_barrier()
    pltpu.sync_copy(acc.at[pl.ds(sid * ROWS_PER_TILE, ROWS_PER_TILE)],
                    out.at[cid, pl.ds(sid * ROWS_PER_TILE, ROWS_PER_TILE)])


_agg_kernel = pl.kernel(
    _agg_body,
    out_type=jax.ShapeDtypeStruct((NC, N_P, F), jnp.float32),
    mesh=_mesh,
    scratch_types=[
        pltpu.VMEM((2, SUB, CHUNK), jnp.int32),
        pltpu.VMEM((2, SUB, CHUNK), jnp.int32),
        pltpu.VMEM((NBUF, CHUNK, F), jnp.float32),
        pltpu.VMEM_SHARED((N_P, F), jnp.float32),
        pltpu.SemaphoreType.DMA((NBUF,)),
        pltpu.SemaphoreType.DMA((NBUF,)),
        pltpu.SemaphoreType.DMA((2,)),
    ],
    compiler_params=pltpu.CompilerParams(use_tc_tiling_on_sc=False),
)


RB = 25600
PB = RB // 4
NPK = N_P // 4
N_BLOCKS = N_P // RB


def _dot(a, b):
    return lax.dot_general(a, b, (((1,), (0,)), ((), ())),
                           preferred_element_type=jnp.float32)


def _mm1_body(x_ref, w_ref, xw_ref):
    xw_ref[...] = _dot(x_ref[...], w_ref[...])


def _mm1(x_packed, W1bd):
    return pl.pallas_call(
        _mm1_body,
        grid=(N_BLOCKS,),
        in_specs=[
            pl.BlockSpec((PB, 244), lambda i: (i, 0)),
            pl.BlockSpec((244, 128), lambda i: (0, 0)),
        ],
        out_specs=pl.BlockSpec((PB, 128), lambda i: (i, 0)),
        out_shape=jax.ShapeDtypeStruct((NPK, 128), jnp.float32),
    )(x_packed, W1bd)


def _stage1_body(xw_ref, d0_ref, d1_ref, s_ref, hs_ref, dis_ref):
    i = pl.program_id(0)
    deg = d0_ref[...] + d1_ref[...] + 1.0
    prow = i * PB + lax.broadcasted_iota(jnp.int32, (PB, 4), 0)
    lane = lax.broadcasted_iota(jnp.int32, (PB, 4), 1)
    node = 4 * prow + lane
    dis4 = jnp.where(node < N_NODES, lax.rsqrt(deg), 0.0)
    dis = _dot(dis4, s_ref[...])
    hs_ref[...] = xw_ref[...] * dis
    dis_ref[...] = dis


def _stage1(xw, d0, d1, sel):
    return pl.pallas_call(
        _stage1_body,
        grid=(N_BLOCKS,),
        in_specs=[
            pl.BlockSpec((PB, 128), lambda i: (i, 0)),
            pl.BlockSpec((PB, 4), lambda i: (i, 0)),
            pl.BlockSpec((PB, 4), lambda i: (i, 0)),
            pl.BlockSpec((4, 128), lambda i: (0, 0)),
        ],
        out_specs=[
            pl.BlockSpec((PB, 128), lambda i: (i, 0)),
            pl.BlockSpec((PB, 128), lambda i: (i, 0)),
        ],
        out_shape=[
            jax.ShapeDtypeStruct((NPK, 128), jnp.float32),
            jax.ShapeDtypeStruct((NPK, 128), jnp.float32),
        ],
    )(xw, d0, d1, sel)


def _stagek_body(p_ref, hs_ref, dis_ref, w_ref, b_ref, out_ref):
    dis = dis_ref[...]
    h = dis * (p_ref[0] + p_ref[1] + hs_ref[...]) + b_ref[...]
    h = jnp.maximum(h, 0.0)
    out_ref[...] = _dot(h, w_ref[...]) * dis


def _stagek(p, hs, dis, Wbd, bt):
    return pl.pallas_call(
        _stagek_body,
        grid=(N_BLOCKS,),
        in_specs=[
            pl.BlockSpec((NC, PB, 128), lambda i: (0, i, 0)),
            pl.BlockSpec((PB, 128), lambda i: (i, 0)),
            pl.BlockSpec((PB, 128), lambda i: (i, 0)),
            pl.BlockSpec((128, 128), lambda i: (0, 0)),
            pl.BlockSpec((1, 128), lambda i: (0, 0)),
        ],
        out_specs=pl.BlockSpec((PB, 128), lambda i: (i, 0)),
        out_shape=jax.ShapeDtypeStruct((NPK, 128), jnp.float32),
    )(p, hs, dis, Wbd, bt)


def _head_body(p_ref, hs_ref, dis_ref, b4_ref, wl1_ref, bl1_ref,
               wl2_ref, bl2_ref, out_ref, acc_ref):
    i = pl.program_id(0)

    @pl.when(i == 0)
    def _():
        acc_ref[...] = jnp.zeros_like(acc_ref)

    h = dis_ref[...] * (p_ref[0] + p_ref[1] + hs_ref[...]) + b4_ref[...]
    h = jnp.maximum(h, 0.0)
    prow = i * PB + lax.broadcasted_iota(jnp.int32, (PB, 128), 0)
    lane4 = lax.broadcasted_iota(jnp.int32, (PB, 128), 1) // F
    node = 4 * prow + lane4
    h = jnp.where(node < N_NODES, h, 0.0)
    acc_ref[...] += jnp.sum(h, axis=0, keepdims=True)

    @pl.when(i == N_BLOCKS - 1)
    def _():
        g = jnp.maximum(_dot(acc_ref[...], wl1_ref[...]) + bl1_ref[...], 0.0)
        out_ref[...] = _dot(g, wl2_ref[...]) + bl2_ref[...]


def _head(p, hs, dis, b4t, Wl1, bl1, Wl2, bl2):
    return pl.pallas_call(
        _head_body,
        grid=(N_BLOCKS,),
        in_specs=[
            pl.BlockSpec((NC, PB, 128), lambda i: (0, i, 0)),
            pl.BlockSpec((PB, 128), lambda i: (i, 0)),
            pl.BlockSpec((PB, 128), lambda i: (i, 0)),
            pl.BlockSpec((1, 128), lambda i: (0, 0)),
            pl.BlockSpec((128, 16), lambda i: (0, 0)),
            pl.BlockSpec((1, 16), lambda i: (0, 0)),
            pl.BlockSpec((16, 3), lambda i: (0, 0)),
            pl.BlockSpec((1, 3), lambda i: (0, 0)),
        ],
        out_specs=pl.BlockSpec((1, 3), lambda i: (0, 0)),
        out_shape=jax.ShapeDtypeStruct((1, 3), jnp.float32),
        scratch_shapes=[pltpu.VMEM((1, 128), jnp.float32)],
    )(p, hs, dis, b4t, Wl1, bl1.reshape(1, 16),
      Wl2, bl2.reshape(1, 3))


def kernel(x, edge_index, W1, b1, W2, b2, W3, b3, W4, b4, Wl1, bl1, Wl2, bl2):
    src = edge_index[0].astype(jnp.int32)
    dst = edge_index[1].astype(jnp.int32)
    npad = E_PAD - N_EDGES
    pad_idx = N_NODES + (jnp.arange(npad, dtype=jnp.int32) % (N_P - N_NODES))
    src2d = jnp.concatenate([src, pad_idx]).reshape(NW, E_PER_TILE_CHUNKS, CHUNK)
    dst2d = jnp.concatenate([dst, pad_idx]).reshape(NW, E_PER_TILE_CHUNKS, CHUNK)

    x_pad = jnp.pad(x, ((0, N_P - N_NODES), (0, 0)))
    ones_hbm = jnp.ones((CHUNK,), jnp.float32)
    zeros1_hbm = jnp.zeros((ROWS_PER_TILE,), jnp.float32)
    zeros2_hbm = jnp.zeros((ROWS_PER_TILE, F), jnp.float32)

    eye4 = jnp.eye(4, dtype=jnp.float32)
    W1bd = jnp.kron(eye4, W1)
    sel = jnp.kron(eye4, jnp.ones((1, F), jnp.float32))
    W2bd = jnp.kron(eye4, W2)
    W3bd = jnp.kron(eye4, W3)
    W4bd = jnp.kron(eye4, W4)
    b1t = jnp.tile(b1, 4).reshape(1, 128)
    b2t = jnp.tile(b2, 4).reshape(1, 128)
    b3t = jnp.tile(b3, 4).reshape(1, 128)
    b4t = jnp.tile(b4, 4).reshape(1, 128)
    Wl1f = jnp.kron(jnp.ones((4, 1), jnp.float32), jnp.eye(F)) @ Wl1

    degp = _deg_kernel(dst2d, ones_hbm, zeros1_hbm)
    xw = _mm1(x_pad.reshape(NPK, 244), W1bd)
    d0 = degp[0].reshape(NPK, 4)
    d1 = degp[1].reshape(NPK, 4)
    hs, dis = _stage1(xw, d0, d1, sel)

    p = _agg_kernel(src2d, dst2d, hs.reshape(N_P, F), zeros2_hbm)
    hs = _stagek(p.reshape(NC, NPK, 128), hs, dis, W2bd, b1t)
    p = _agg_kernel(src2d, dst2d, hs.reshape(N_P, F), zeros2_hbm)
    hs = _stagek(p.reshape(NC, NPK, 128), hs, dis, W3bd, b2t)
    p = _agg_kernel(src2d, dst2d, hs.reshape(N_P, F), zeros2_hbm)
    hs = _stagek(p.reshape(NC, NPK, 128), hs, dis, W4bd, b3t)
    p = _agg_kernel(src2d, dst2d, hs.reshape(N_P, F), zeros2_hbm)
    g = _head(p.reshape(NC, NPK, 128), hs, dis, b4t, Wl1f, bl1, Wl2, bl2)
    return g.reshape(3)


# speedup vs baseline: 1.0004x; 1.0004x over previous
import functools

import jax
import jax.numpy as jnp
from jax import lax
from jax.experimental import pallas as pl
from jax.experimental.pallas import tpu as pltpu
from jax.experimental.pallas import tpu_sc as plsc

N_NODES = 50000
N_EDGES = 800000

NC = 2
NS = 16
NW = NC * NS

CHUNK = 128
E_PER_TILE_CHUNKS = 200
E_PER_TILE = E_PER_TILE_CHUNKS * CHUNK
E_PAD = E_PER_TILE * NW
N_P = 51200
ROWS_PER_TILE = N_P // NS
F = 32

_mesh = plsc.VectorSubcoreMesh(core_axis_name="c", subcore_axis_name="s")


DEG_K = 8


def _deg_body(dst2d, ones_hbm, zeros_hbm, out, dst_v, ones_v, acc, sem):
    cid = lax.axis_index("c")
    sid = lax.axis_index("s")
    w = cid * NS + sid
    pltpu.sync_copy(dst2d.at[w], dst_v)
    pltpu.sync_copy(ones_hbm, ones_v)
    pltpu.sync_copy(zeros_hbm, acc.at[pl.ds(sid * ROWS_PER_TILE, ROWS_PER_TILE)])
    plsc.subcore_barrier()

    def body(g, carry):
        for i in range(DEG_K):
            pltpu.async_copy(ones_v, acc.at[dst_v.at[DEG_K * g + i]], sem,
                             add=True)
        for i in range(DEG_K):
            pltpu.make_async_copy(ones_v, acc.at[dst_v.at[0]], sem).wait()
        return carry

    lax.fori_loop(0, E_PER_TILE_CHUNKS // DEG_K, body, 0, unroll=False)
    plsc.subcore_barrier()
    pltpu.sync_copy(acc.at[pl.ds(sid * ROWS_PER_TILE, ROWS_PER_TILE)],
                    out.at[cid, pl.ds(sid * ROWS_PER_TILE, ROWS_PER_TILE)])


_deg_kernel = pl.kernel(
    _deg_body,
    out_type=jax.ShapeDtypeStruct((NC, N_P), jnp.float32),
    mesh=_mesh,
    scratch_types=[
        pltpu.VMEM((E_PER_TILE_CHUNKS, CHUNK), jnp.int32),
        pltpu.VMEM((CHUNK,), jnp.float32),
        pltpu.VMEM_SHARED((N_P,), jnp.float32),
        pltpu.SemaphoreType.DMA,
    ],
    compiler_params=pltpu.CompilerParams(skip_device_barrier=True),
)


SUB = 20
SUP = E_PER_TILE_CHUNKS // SUB
NBUF = 4
NZC = ROWS_PER_TILE // CHUNK


def _agg_body(src2d, dst2d, hs, zeros_hbm, out,
              src_v, dst_v, bufs, acc, gsems, ssems, isems):
    cid = lax.axis_index("c")
    sid = lax.axis_index("s")
    w = cid * NS + sid

    def istart(s, slab):
        pltpu.async_copy(src2d.at[w, pl.ds(s * SUB, SUB)], src_v.at[slab],
                         isems.at[slab])
        pltpu.async_copy(dst2d.at[w, pl.ds(s * SUB, SUB)], dst_v.at[slab],
                         isems.at[slab])

    def iwait(slab):
        pltpu.make_async_copy(src2d.at[w, pl.ds(0, SUB)], src_v.at[slab],
                              isems.at[slab]).wait()
        pltpu.make_async_copy(dst2d.at[w, pl.ds(0, SUB)], dst_v.at[slab],
                              isems.at[slab]).wait()

    istart(0, 0)
    pltpu.sync_copy(zeros_hbm,
                    acc.at[pl.ds(sid * ROWS_PER_TILE, ROWS_PER_TILE)])
    plsc.subcore_barrier()

    def gstart(slab, j, i):
        pltpu.async_copy(hs.at[src_v.at[slab, j]], bufs.at[i], gsems.at[i])

    def gwait(i):
        pltpu.make_async_copy(hs.at[src_v.at[0, 0]], bufs.at[i],
                              gsems.at[i]).wait()

    def sstart(slab, j, i):
        pltpu.async_copy(bufs.at[i], acc.at[dst_v.at[slab, j]],
                         ssems.at[i], add=True)

    def swait(i):
        pltpu.make_async_copy(bufs.at[i], acc.at[dst_v.at[0, 0]],
                              ssems.at[i]).wait()

    def super_body(s, carry):
        slab = s % 2
        iwait(slab)

        @pl.when(s + 1 < SUP)
        def _():
            istart(s + 1, 1 - slab)

        for i in range(NBUF):
            gstart(slab, i, i)

        def body(g, carry2):
            j = NBUF * g
            for i in range(NBUF):
                gwait(i)
                sstart(slab, j + i, i)
            for i in range(NBUF):
                swait(i)
                gstart(slab, j + NBUF + i, i)
            return carry2

        lax.fori_loop(0, SUB // NBUF - 1, body, 0, unroll=False)
        j = SUB - NBUF
        for i in range(NBUF):
            gwait(i)
            sstart(slab, j + i, i)
        for i in range(NBUF):
            swait(i)
        return carry

    lax.fori_loop(0, SUP, super_body, 0, unroll=False)

    plsc.subcore_barrier()
    pltpu.sync_copy(acc.at[pl.ds(sid * ROWS_PER_TILE, ROWS_PER_TILE)],
                    out.at[cid, pl.ds(sid * ROWS_PER_TILE, ROWS_PER_TILE)])


_agg_kernel = pl.kernel(
    _agg_body,
    out_type=jax.ShapeDtypeStruct((NC, N_P, F), jnp.float32),
    mesh=_mesh,
    scratch_types=[
        pltpu.VMEM((2, SUB, CHUNK), jnp.int32),
        pltpu.VMEM((2, SUB, CHUNK), jnp.int32),
        pltpu.VMEM((NBUF, CHUNK, F), jnp.float32),
        pltpu.VMEM_SHARED((N_P, F), jnp.float32),
        pltpu.SemaphoreType.DMA((NBUF,)),
        pltpu.SemaphoreType.DMA((NBUF,)),
        pltpu.SemaphoreType.DMA((2,)),
    ],
    compiler_params=pltpu.CompilerParams(use_tc_tiling_on_sc=False,
                                         skip_device_barrier=True),
)


RB = 12800
PB = RB // 4
NPK = N_P // 4
N_BLOCKS = N_P // RB


def _dot(a, b):
    return lax.dot_general(a, b, (((1,), (0,)), ((), ())),
                           preferred_element_type=jnp.float32)


def _mm1_body(x_ref, w_ref, xw_ref):
    xw_ref[...] = _dot(x_ref[...], w_ref[...])


def _mm1(x_packed, W1bd):
    return pl.pallas_call(
        _mm1_body,
        grid=(N_BLOCKS,),
        in_specs=[
            pl.BlockSpec((PB, 244), lambda i: (i, 0)),
            pl.BlockSpec((244, 128), lambda i: (0, 0)),
        ],
        out_specs=pl.BlockSpec((PB, 128), lambda i: (i, 0)),
        out_shape=jax.ShapeDtypeStruct((NPK, 128), jnp.float32),
    )(x_packed, W1bd)


def _stage1_body(xw_ref, d0_ref, d1_ref, s_ref, hs_ref, dis_ref):
    i = pl.program_id(0)
    deg = d0_ref[...] + d1_ref[...] + 1.0
    prow = i * PB + lax.broadcasted_iota(jnp.int32, (PB, 4), 0)
    lane = lax.broadcasted_iota(jnp.int32, (PB, 4), 1)
    node = 4 * prow + lane
    dis4 = jnp.where(node < N_NODES, lax.rsqrt(deg), 0.0)
    dis = _dot(dis4, s_ref[...])
    hs_ref[...] = xw_ref[...] * dis
    dis_ref[...] = dis


def _stage1(xw, d0, d1, sel):
    return pl.pallas_call(
        _stage1_body,
        grid=(N_BLOCKS,),
        in_specs=[
            pl.BlockSpec((PB, 128), lambda i: (i, 0)),
            pl.BlockSpec((PB, 4), lambda i: (i, 0)),
            pl.BlockSpec((PB, 4), lambda i: (i, 0)),
            pl.BlockSpec((4, 128), lambda i: (0, 0)),
        ],
        out_specs=[
            pl.BlockSpec((PB, 128), lambda i: (i, 0)),
            pl.BlockSpec((PB, 128), lambda i: (i, 0)),
        ],
        out_shape=[
            jax.ShapeDtypeStruct((NPK, 128), jnp.float32),
            jax.ShapeDtypeStruct((NPK, 128), jnp.float32),
        ],
    )(xw, d0, d1, sel)


def _stagek_body(p_ref, hs_ref, dis_ref, w_ref, b_ref, out_ref):
    dis = dis_ref[...]
    h = dis * (p_ref[0] + p_ref[1] + hs_ref[...]) + b_ref[...]
    h = jnp.maximum(h, 0.0)
    out_ref[...] = _dot(h, w_ref[...]) * dis


def _stagek(p, hs, dis, Wbd, bt):
    return pl.pallas_call(
        _stagek_body,
        grid=(N_BLOCKS,),
        in_specs=[
            pl.BlockSpec((NC, PB, 128), lambda i: (0, i, 0)),
            pl.BlockSpec((PB, 128), lambda i: (i, 0)),
            pl.BlockSpec((PB, 128), lambda i: (i, 0)),
            pl.BlockSpec((128, 128), lambda i: (0, 0)),
            pl.BlockSpec((1, 128), lambda i: (0, 0)),
        ],
        out_specs=pl.BlockSpec((PB, 128), lambda i: (i, 0)),
        out_shape=jax.ShapeDtypeStruct((NPK, 128), jnp.float32),
    )(p, hs, dis, Wbd, bt)


def _head_body(p_ref, hs_ref, dis_ref, b4_ref, wl1_ref, bl1_ref,
               wl2_ref, bl2_ref, out_ref, acc_ref):
    i = pl.program_id(0)

    @pl.when(i == 0)
    def _():
        acc_ref[...] = jnp.zeros_like(acc_ref)

    h = dis_ref[...] * (p_ref[0] + p_ref[1] + hs_ref[...]) + b4_ref[...]
    h = jnp.maximum(h, 0.0)
    prow = i * PB + lax.broadcasted_iota(jnp.int32, (PB, 128), 0)
    lane4 = lax.broadcasted_iota(jnp.int32, (PB, 128), 1) // F
    node = 4 * prow + lane4
    h = jnp.where(node < N_NODES, h, 0.0)
    acc_ref[...] += jnp.sum(h, axis=0, keepdims=True)

    @pl.when(i == N_BLOCKS - 1)
    def _():
        g = jnp.maximum(_dot(acc_ref[...], wl1_ref[...]) + bl1_ref[...], 0.0)
        out_ref[...] = _dot(g, wl2_ref[...]) + bl2_ref[...]


def _head(p, hs, dis, b4t, Wl1, bl1, Wl2, bl2):
    return pl.pallas_call(
        _head_body,
        grid=(N_BLOCKS,),
        in_specs=[
            pl.BlockSpec((NC, PB, 128), lambda i: (0, i, 0)),
            pl.BlockSpec((PB, 128), lambda i: (i, 0)),
            pl.BlockSpec((PB, 128), lambda i: (i, 0)),
            pl.BlockSpec((1, 128), lambda i: (0, 0)),
            pl.BlockSpec((128, 16), lambda i: (0, 0)),
            pl.BlockSpec((1, 16), lambda i: (0, 0)),
            pl.BlockSpec((16, 3), lambda i: (0, 0)),
            pl.BlockSpec((1, 3), lambda i: (0, 0)),
        ],
        out_specs=pl.BlockSpec((1, 3), lambda i: (0, 0)),
        out_shape=jax.ShapeDtypeStruct((1, 3), jnp.float32),
        scratch_shapes=[pltpu.VMEM((1, 128), jnp.float32)],
    )(p, hs, dis, b4t, Wl1, bl1.reshape(1, 16),
      Wl2, bl2.reshape(1, 3))


def kernel(x, edge_index, W1, b1, W2, b2, W3, b3, W4, b4, Wl1, bl1, Wl2, bl2):
    src = edge_index[0].astype(jnp.int32)
    dst = edge_index[1].astype(jnp.int32)
    npad = E_PAD - N_EDGES
    pad_idx = N_NODES + (jnp.arange(npad, dtype=jnp.int32) % (N_P - N_NODES))
    src2d = jnp.concatenate([src, pad_idx]).reshape(NW, E_PER_TILE_CHUNKS, CHUNK)
    dst2d = jnp.concatenate([dst, pad_idx]).reshape(NW, E_PER_TILE_CHUNKS, CHUNK)

    x_pad = jnp.pad(x, ((0, N_P - N_NODES), (0, 0)))
    ones_hbm = jnp.ones((CHUNK,), jnp.float32)
    zeros1_hbm = jnp.zeros((ROWS_PER_TILE,), jnp.float32)
    zeros2_hbm = jnp.zeros((ROWS_PER_TILE, F), jnp.float32)

    eye4 = jnp.eye(4, dtype=jnp.float32)
    W1bd = jnp.kron(eye4, W1)
    sel = jnp.kron(eye4, jnp.ones((1, F), jnp.float32))
    W2bd = jnp.kron(eye4, W2)
    W3bd = jnp.kron(eye4, W3)
    W4bd = jnp.kron(eye4, W4)
    b1t = jnp.tile(b1, 4).reshape(1, 128)
    b2t = jnp.tile(b2, 4).reshape(1, 128)
    b3t = jnp.tile(b3, 4).reshape(1, 128)
    b4t = jnp.tile(b4, 4).reshape(1, 128)
    Wl1f = jnp.kron(jnp.ones((4, 1), jnp.float32), jnp.eye(F)) @ Wl1

    degp = _deg_kernel(dst2d, ones_hbm, zeros1_hbm)
    xw = _mm1(x_pad.reshape(NPK, 244), W1bd)
    d0 = degp[0].reshape(NPK, 4)
    d1 = degp[1].reshape(NPK, 4)
    hs, dis = _stage1(xw, d0, d1, sel)

    p = _agg_kernel(src2d, dst2d, hs.reshape(N_P, F), zeros2_hbm)
    hs = _stagek(p.reshape(NC, NPK, 128), hs, dis, W2bd, b1t)
    p = _agg_kernel(src2d, dst2d, hs.reshape(N_P, F), zeros2_hbm)
    hs = _stagek(p.reshape(NC, NPK, 128), hs, dis, W3bd, b2t)
    p = _agg_kernel(src2d, dst2d, hs.reshape(N_P, F), zeros2_hbm)
    hs = _stagek(p.reshape(NC, NPK, 128), hs, dis, W4bd, b3t)
    p = _agg_kernel(src2d, dst2d, hs.reshape(N_P, F), zeros2_hbm)
    g = _head(p.reshape(NC, NPK, 128), hs, dis, b4t, Wl1f, bl1, Wl2, bl2)
    return g.reshape(3)
